# baseline (speedup 1.0000x reference)
import jax
import jax.numpy as jnp
from jax.experimental import pallas as pl
from jax.experimental.pallas import tpu as pltpu

B, N, C, W, H = 8, 16, 64, 32, 32
TS = 64
ROWS = 2
CH = (B * N) // ROWS
NBUF = 8


def _assemble_kernel(x_ref, t_ref, emb_ref, out_ref, stage_ref, insem, outsem):
    for k in range(NBUF):
        for r in range(ROWS):
            n = ROWS * k + r
            row = emb_ref[t_ref[n], :]
            stage_ref[k, r, C:] = jax.lax.broadcast_in_dim(row, (TS, W, H), (0,))

    def in_copy(c):
        k = c % NBUF
        return pltpu.make_async_copy(
            x_ref.at[pl.ds(c * ROWS, ROWS)], stage_ref.at[k, :, 0:C], insem.at[k])

    def out_copy(c):
        k = c % NBUF
        return pltpu.make_async_copy(
            stage_ref.at[k], out_ref.at[pl.ds(c * ROWS, ROWS)], outsem.at[k])

    ins = {}
    outs = {}
    for c in range(NBUF):
        ins[c] = in_copy(c)
        ins[c].start()
    for c in range(CH):
        if c >= NBUF:
            outs[c - NBUF].wait()
            ins[c] = in_copy(c)
            ins[c].start()
        ins[c].wait()
        outs[c] = out_copy(c)
        outs[c].start()
    for c in range(CH - NBUF, CH):
        outs[c].wait()


def kernel(x, t, emb_table):
    x3 = x.reshape(B * N, C, W, H)
    out = pl.pallas_call(
        _assemble_kernel,
        in_specs=[
            pl.BlockSpec(memory_space=pl.ANY),
            pl.BlockSpec(memory_space=pltpu.SMEM),
            pl.BlockSpec(memory_space=pltpu.VMEM),
        ],
        out_specs=pl.BlockSpec(memory_space=pl.ANY),
        out_shape=jax.ShapeDtypeStruct((B * N, C + TS, W, H), x.dtype),
        scratch_shapes=[
            pltpu.VMEM((NBUF, ROWS, C + TS, W, H), x.dtype),
            pltpu.SemaphoreType.DMA((NBUF,)),
            pltpu.SemaphoreType.DMA((NBUF,)),
        ],
    )(x3, t.astype(jnp.int32), emb_table)
    return out.reshape(B, N, C + TS, W, H)
